# baseline (speedup 1.0000x reference)
import functools

import jax
import jax.numpy as jnp
import numpy as np
from jax.experimental import pallas as pl
from jax.experimental.pallas import tpu as pltpu

_B, _N, _IN, _OUT, _H = 8, 1024, 128, 64, 4


def _fused_gat_kernel(x_ref, adj_ref, w1_ref, a1s_ref, a1d_ref, wout_ref,
                      aout_ref, out_ref, adjf_ref):
    x = x_ref[0]
    adjf_ref[...] = (adj_ref[0] > 0).astype(jnp.bfloat16)
    adjf = adjf_ref[...]

    wh = jnp.dot(x, w1_ref[...], preferred_element_type=jnp.float32)
    es = jnp.dot(wh, a1s_ref[...], preferred_element_type=jnp.float32)
    ed = jnp.dot(wh, a1d_ref[...], preferred_element_type=jnp.float32)
    whb = wh.astype(jnp.bfloat16)
    ones = jnp.ones((_N, 1), jnp.bfloat16)

    u1 = jnp.exp(es).astype(jnp.bfloat16)
    u2 = jnp.exp(0.2 * es).astype(jnp.bfloat16)
    v1 = jnp.exp(ed).T.astype(jnp.bfloat16)
    v2 = jnp.exp(0.2 * ed).T.astype(jnp.bfloat16)

    acc = jnp.zeros((_N, _OUT), jnp.float32)
    for h in range(_H):
        t1 = u1[:, h:h + 1] * v1[h:h + 1, :]
        t2 = u2[:, h:h + 1] * v2[h:h + 1, :]
        p = jnp.maximum(t1, t2) * adjf
        whc = jnp.concatenate([whb[:, h * _OUT:(h + 1) * _OUT], ones], axis=1)
        hps = jnp.dot(p, whc, preferred_element_type=jnp.float32)
        hp = hps[:, :_OUT] * (1.0 / hps[:, _OUT:_OUT + 1])
        acc = acc + jnp.where(hp > 0, hp, jnp.exp(hp) - 1.0)

    x2 = acc * (1.0 / _H)
    wh2 = jnp.dot(x2, wout_ref[...], preferred_element_type=jnp.float32)
    e2 = jnp.dot(wh2, aout_ref[...], preferred_element_type=jnp.float32)
    u1o = jnp.exp(e2[:, 0:1]).astype(jnp.bfloat16)
    u2o = jnp.exp(0.2 * e2[:, 0:1]).astype(jnp.bfloat16)
    v1o = jnp.exp(e2[:, 1:2]).T.astype(jnp.bfloat16)
    v2o = jnp.exp(0.2 * e2[:, 1:2]).T.astype(jnp.bfloat16)
    t1 = u1o * v1o
    t2 = u2o * v2o
    p = jnp.maximum(t1, t2) * adjf
    whc2 = jnp.concatenate([wh2.astype(jnp.bfloat16), ones], axis=1)
    os = jnp.dot(p, whc2, preferred_element_type=jnp.float32)
    o = os[:, :_OUT] * (1.0 / os[:, _OUT:_OUT + 1])
    out_ref[0] = jnp.maximum(o, 0.0)


def _gat_call(input_feature, adj, w1r, a1s, a1d, W_out, aout2, *, nb):
    return pl.pallas_call(
        _fused_gat_kernel,
        grid=(nb,),
        in_specs=[
            pl.BlockSpec((1, _N, _IN), lambda b: (b, 0, 0)),
            pl.BlockSpec((1, _N, _N), lambda b: (b, 0, 0)),
            pl.BlockSpec((_IN, _H * _OUT), lambda b: (0, 0)),
            pl.BlockSpec((_H * _OUT, _H), lambda b: (0, 0)),
            pl.BlockSpec((_H * _OUT, _H), lambda b: (0, 0)),
            pl.BlockSpec((_OUT, _OUT), lambda b: (0, 0)),
            pl.BlockSpec((_OUT, 2), lambda b: (0, 0)),
        ],
        out_specs=pl.BlockSpec((1, _N, _OUT), lambda b: (b, 0, 0)),
        out_shape=jax.ShapeDtypeStruct((nb, _N, _OUT), jnp.float32),
        scratch_shapes=[
            pltpu.VMEM((_N, _N), jnp.bfloat16),
        ],
    )(input_feature, adj, w1r, a1s, a1d, W_out, aout2)


def kernel(input_feature, adj, W1, a1, W_out, a_out):
    w1r = jnp.transpose(W1, (1, 0, 2)).reshape(_IN, _H * _OUT)
    a_src = a1[:, :_OUT, 0]
    a_dst = a1[:, _OUT:, 0]
    eye = jnp.eye(_H, dtype=jnp.float32)
    a1s = (eye[:, None, :] * a_src[:, :, None]).reshape(_H * _OUT, _H)
    a1d = (eye[:, None, :] * a_dst[:, :, None]).reshape(_H * _OUT, _H)
    aout2 = a_out.reshape(2, _OUT).T

    devs = jax.devices()
    n_shards = 2 if len(devs) >= 2 and _B % 2 == 0 else 1
    if n_shards == 1:
        return _gat_call(input_feature, adj, w1r, a1s, a1d, W_out, aout2,
                         nb=_B)
    mesh = jax.sharding.Mesh(np.array(devs[:n_shards]), ("d",))
    pspec = jax.sharding.PartitionSpec
    fn = jax.shard_map(
        functools.partial(_gat_call, nb=_B // n_shards),
        mesh=mesh,
        in_specs=(pspec("d"), pspec("d"), pspec(), pspec(), pspec(),
                  pspec(), pspec()),
        out_specs=pspec("d"),
        check_vma=False,
    )
    return fn(input_feature, adj, w1r, a1s, a1d, W_out, aout2)


# speedup vs baseline: 9.6175x; 9.6175x over previous
import jax
import jax.numpy as jnp
from jax.experimental import pallas as pl
from jax.experimental.pallas import tpu as pltpu

_B, _N, _IN, _OUT, _H = 8, 1024, 128, 64, 4


def _fused_gat_kernel(x_ref, adj_ref, w1_ref, a1s_ref, a1d_ref, wout_ref,
                      aout_ref, out_ref, adjf_ref):
    x = x_ref[0]
    adjf_ref[...] = adj_ref[0].astype(jnp.bfloat16)
    adjf = adjf_ref[...]

    wh = jnp.dot(x, w1_ref[...], preferred_element_type=jnp.float32)
    es = jnp.dot(wh, a1s_ref[...], preferred_element_type=jnp.float32)
    ed = jnp.dot(wh, a1d_ref[...], preferred_element_type=jnp.float32)
    whb = wh.astype(jnp.bfloat16)
    ones = jnp.ones((_N, 1), jnp.bfloat16)

    u1 = jnp.exp(es).astype(jnp.bfloat16)
    u2 = jnp.exp(0.2 * es).astype(jnp.bfloat16)
    v1 = jnp.exp(ed).T.astype(jnp.bfloat16)
    v2 = jnp.exp(0.2 * ed).T.astype(jnp.bfloat16)

    acc = jnp.zeros((_N, _OUT), jnp.float32)
    _CB = 256
    for h in range(_H):
        whc = jnp.concatenate([whb[:, h * _OUT:(h + 1) * _OUT], ones], axis=1)
        u1c = u1[:, h:h + 1]
        u2c = u2[:, h:h + 1]
        hps = jnp.zeros((_N, _OUT + 1), jnp.float32)
        for kb in range(_N // _CB):
            sl = slice(kb * _CB, (kb + 1) * _CB)
            t1 = u1c * v1[h:h + 1, sl]
            t2 = u2c * v2[h:h + 1, sl]
            pblk = jnp.maximum(t1, t2) * adjf[:, sl]
            hps = hps + jnp.dot(pblk, whc[sl, :],
                                preferred_element_type=jnp.float32)
        hp = hps[:, :_OUT] * (1.0 / hps[:, _OUT:_OUT + 1])
        acc = acc + jnp.where(hp > 0, hp, jnp.exp(hp) - 1.0)

    x2 = acc * (1.0 / _H)
    wh2 = jnp.dot(x2, wout_ref[...], preferred_element_type=jnp.float32)
    e2 = jnp.dot(wh2, aout_ref[...], preferred_element_type=jnp.float32)
    u1o = jnp.exp(e2[:, 0:1]).astype(jnp.bfloat16)
    u2o = jnp.exp(0.2 * e2[:, 0:1]).astype(jnp.bfloat16)
    v1o = jnp.exp(e2[:, 1:2]).T.astype(jnp.bfloat16)
    v2o = jnp.exp(0.2 * e2[:, 1:2]).T.astype(jnp.bfloat16)
    t1 = u1o * v1o
    t2 = u2o * v2o
    p = jnp.maximum(t1, t2) * adjf
    whc2 = jnp.concatenate([wh2.astype(jnp.bfloat16), ones], axis=1)
    os = jnp.dot(p, whc2, preferred_element_type=jnp.float32)
    o = os[:, :_OUT] * (1.0 / os[:, _OUT:_OUT + 1])
    out_ref[0] = jnp.maximum(o, 0.0)


def _gat_call(input_feature, adj, w1r, a1s, a1d, W_out, aout2, *, nb):
    return pl.pallas_call(
        _fused_gat_kernel,
        grid=(nb,),
        in_specs=[
            pl.BlockSpec((1, _N, _IN), lambda b: (b, 0, 0)),
            pl.BlockSpec((1, _N, _N), lambda b: (b, 0, 0)),
            pl.BlockSpec((_IN, _H * _OUT), lambda b: (0, 0)),
            pl.BlockSpec((_H * _OUT, _H), lambda b: (0, 0)),
            pl.BlockSpec((_H * _OUT, _H), lambda b: (0, 0)),
            pl.BlockSpec((_OUT, _OUT), lambda b: (0, 0)),
            pl.BlockSpec((_OUT, 2), lambda b: (0, 0)),
        ],
        out_specs=pl.BlockSpec((1, _N, _OUT), lambda b: (b, 0, 0)),
        out_shape=jax.ShapeDtypeStruct((nb, _N, _OUT), jnp.float32),
        scratch_shapes=[
            pltpu.VMEM((_N, _N), jnp.bfloat16),
        ],
    )(input_feature, adj, w1r, a1s, a1d, W_out, aout2)


def kernel(input_feature, adj, W1, a1, W_out, a_out):
    w1r = jnp.transpose(W1, (1, 0, 2)).reshape(_IN, _H * _OUT)
    a_src = a1[:, :_OUT, 0]
    a_dst = a1[:, _OUT:, 0]
    eye = jnp.eye(_H, dtype=jnp.float32)
    a1s = (eye[:, None, :] * a_src[:, :, None]).reshape(_H * _OUT, _H)
    a1d = (eye[:, None, :] * a_dst[:, :, None]).reshape(_H * _OUT, _H)
    aout2 = a_out.reshape(2, _OUT).T

    return _gat_call(input_feature, adj, w1r, a1s, a1d, W_out, aout2, nb=_B)


# speedup vs baseline: 10.3074x; 1.0717x over previous
import jax
import jax.numpy as jnp
from jax.experimental import pallas as pl
from jax.experimental.pallas import tpu as pltpu

_B, _N, _IN, _OUT, _H = 8, 1024, 128, 64, 4


def _fused_gat_kernel(x_ref, adj_ref, w1_ref, a1s_ref, a1d_ref, wout_ref,
                      aout_ref, out_ref, adjf_ref):
    for g in range(x_ref.shape[0]):
        _one_graph(x_ref, adj_ref, w1_ref, a1s_ref, a1d_ref, wout_ref,
                   aout_ref, out_ref, adjf_ref, g)


def _one_graph(x_ref, adj_ref, w1_ref, a1s_ref, a1d_ref, wout_ref,
               aout_ref, out_ref, adjf_ref, g):
    x = x_ref[g]
    adjf_ref[...] = adj_ref[g].astype(jnp.bfloat16)
    adjf = adjf_ref[...]

    wh = jnp.dot(x, w1_ref[...], preferred_element_type=jnp.float32)
    es = jnp.dot(wh, a1s_ref[...], preferred_element_type=jnp.float32)
    ed = jnp.dot(wh, a1d_ref[...], preferred_element_type=jnp.float32)
    whb = wh.astype(jnp.bfloat16)
    ones = jnp.ones((_N, 1), jnp.bfloat16)

    u1 = jnp.exp(es).astype(jnp.bfloat16)
    u2 = jnp.exp(0.2 * es).astype(jnp.bfloat16)
    v1 = jnp.exp(ed).T.astype(jnp.bfloat16)
    v2 = jnp.exp(0.2 * ed).T.astype(jnp.bfloat16)

    acc = jnp.zeros((_N, _OUT), jnp.float32)
    _CB = 256
    for h in range(_H):
        whc = jnp.concatenate([whb[:, h * _OUT:(h + 1) * _OUT], ones], axis=1)
        u1c = u1[:, h:h + 1]
        u2c = u2[:, h:h + 1]
        hps = jnp.zeros((_N, _OUT + 1), jnp.float32)
        for kb in range(_N // _CB):
            sl = slice(kb * _CB, (kb + 1) * _CB)
            t1 = u1c * v1[h:h + 1, sl]
            t2 = u2c * v2[h:h + 1, sl]
            pblk = jnp.maximum(t1, t2) * adjf[:, sl]
            hps = hps + jnp.dot(pblk, whc[sl, :],
                                preferred_element_type=jnp.float32)
        hp = hps[:, :_OUT] * (1.0 / hps[:, _OUT:_OUT + 1])
        acc = acc + jnp.where(hp > 0, hp, jnp.exp(hp) - 1.0)

    x2 = acc * (1.0 / _H)
    wh2 = jnp.dot(x2, wout_ref[...], preferred_element_type=jnp.float32)
    e2 = jnp.dot(wh2, aout_ref[...], preferred_element_type=jnp.float32)
    u1o = jnp.exp(e2[:, 0:1]).astype(jnp.bfloat16)
    u2o = jnp.exp(0.2 * e2[:, 0:1]).astype(jnp.bfloat16)
    v1o = jnp.exp(e2[:, 1:2]).T.astype(jnp.bfloat16)
    v2o = jnp.exp(0.2 * e2[:, 1:2]).T.astype(jnp.bfloat16)
    t1 = u1o * v1o
    t2 = u2o * v2o
    p = jnp.maximum(t1, t2) * adjf
    whc2 = jnp.concatenate([wh2.astype(jnp.bfloat16), ones], axis=1)
    os = jnp.dot(p, whc2, preferred_element_type=jnp.float32)
    o = os[:, :_OUT] * (1.0 / os[:, _OUT:_OUT + 1])
    out_ref[g] = jnp.maximum(o, 0.0)


def _gat_call(input_feature, adj, w1r, a1s, a1d, W_out, aout2, *, nb, gpb=2):
    return pl.pallas_call(
        _fused_gat_kernel,
        grid=(nb // gpb,),
        in_specs=[
            pl.BlockSpec((gpb, _N, _IN), lambda b: (b, 0, 0)),
            pl.BlockSpec((gpb, _N, _N), lambda b: (b, 0, 0)),
            pl.BlockSpec((_IN, _H * _OUT), lambda b: (0, 0)),
            pl.BlockSpec((_H * _OUT, _H), lambda b: (0, 0)),
            pl.BlockSpec((_H * _OUT, _H), lambda b: (0, 0)),
            pl.BlockSpec((_OUT, _OUT), lambda b: (0, 0)),
            pl.BlockSpec((_OUT, 2), lambda b: (0, 0)),
        ],
        out_specs=pl.BlockSpec((gpb, _N, _OUT), lambda b: (b, 0, 0)),
        out_shape=jax.ShapeDtypeStruct((nb, _N, _OUT), jnp.float32),
        scratch_shapes=[
            pltpu.VMEM((_N, _N), jnp.bfloat16),
        ],
    )(input_feature, adj, w1r, a1s, a1d, W_out, aout2)


def kernel(input_feature, adj, W1, a1, W_out, a_out):
    w1r = jnp.transpose(W1, (1, 0, 2)).reshape(_IN, _H * _OUT)
    a_src = a1[:, :_OUT, 0]
    a_dst = a1[:, _OUT:, 0]
    eye = jnp.eye(_H, dtype=jnp.float32)
    a1s = (eye[:, None, :] * a_src[:, :, None]).reshape(_H * _OUT, _H)
    a1d = (eye[:, None, :] * a_dst[:, :, None]).reshape(_H * _OUT, _H)
    aout2 = a_out.reshape(2, _OUT).T

    return _gat_call(input_feature, adj, w1r, a1s, a1d, W_out, aout2, nb=_B)
